# baseline (speedup 1.0000x reference)
import functools

import jax
import jax.numpy as jnp
from jax import lax
from jax.experimental import pallas as pl
from jax.experimental.pallas import tpu as pltpu
from jax.experimental.pallas import tpu_sc as plsc

N_TOK = 8192
HID = 2048
N_EXP = 16
BM = 512

NW = 16
TOK_PER_W = N_TOK // NW


def _router_gate_body(xa_ref, xb_ref, w_ref, gate_ref):
    dn = (((1,), (1,)), ((), ()))
    logits = lax.dot_general(
        w_ref[:, : HID // 2], xa_ref[...],
        dimension_numbers=dn, preferred_element_type=jnp.float32,
    ) + lax.dot_general(
        w_ref[:, HID // 2 :], xb_ref[...],
        dimension_numbers=dn, preferred_element_type=jnp.float32,
    )
    m = jnp.max(logits, axis=0, keepdims=True)
    e = jnp.exp(logits - m)
    gate_ref[...] = e / jnp.sum(e, axis=0, keepdims=True)


def _gate_tc(x, w):
    return pl.pallas_call(
        _router_gate_body,
        grid=(N_TOK // BM,),
        in_specs=[
            pl.BlockSpec((BM, HID // 2), lambda i: (i, 0)),
            pl.BlockSpec((BM, HID // 2), lambda i: (i, 1)),
            pl.BlockSpec((N_EXP, HID), lambda i: (0, 0)),
        ],
        out_specs=pl.BlockSpec((N_EXP, BM), lambda i: (0, i)),
        out_shape=jax.ShapeDtypeStruct((N_EXP, N_TOK), jnp.float32),
    )(x, x, w)


def _top2_sc_body(gate_hbm, val_hbm, idx_hbm, gate_v, outv, outi):
    wid = lax.axis_index("s") * 1 + lax.axis_index("c")
    base = wid * TOK_PER_W

    pltpu.sync_copy(gate_hbm.at[:, pl.ds(base, TOK_PER_W)], gate_v)

    def grp_body(g, carry):
        sl = pl.ds(g * 16, 16)
        vals = [gate_v[e, sl] for e in range(N_EXP)]
        m1 = vals[0]
        i1 = jnp.zeros((16,), jnp.int32)
        for e in range(1, N_EXP):
            c = vals[e] > m1
            m1 = jnp.where(c, vals[e], m1)
            i1 = jnp.where(c, e, i1)
        m2 = jnp.full((16,), -1.0, jnp.float32)
        i2 = jnp.zeros((16,), jnp.int32)
        for e in range(N_EXP):
            ve = jnp.where(i1 == e, -1.0, vals[e])
            c = ve > m2
            m2 = jnp.where(c, ve, m2)
            i2 = jnp.where(c, e, i2)
        outv[0, sl] = m1
        outv[1, sl] = m2
        outi[0, sl] = i1
        outi[1, sl] = i2
        return carry

    lax.fori_loop(0, TOK_PER_W // 16, grp_body, 0, unroll=2)

    pltpu.sync_copy(outv, val_hbm.at[:, pl.ds(base, TOK_PER_W)])
    pltpu.sync_copy(outi, idx_hbm.at[:, pl.ds(base, TOK_PER_W)])


def _top2_sc(gate_t):
    mesh = plsc.VectorSubcoreMesh(core_axis_name="c", subcore_axis_name="s", num_cores=1)
    f = functools.partial(
        pl.kernel,
        out_type=[
            jax.ShapeDtypeStruct((2, N_TOK), jnp.float32),
            jax.ShapeDtypeStruct((2, N_TOK), jnp.int32),
        ],
        mesh=mesh,
        compiler_params=pltpu.CompilerParams(needs_layout_passes=False),
        scratch_types=[
            pltpu.VMEM((N_EXP, TOK_PER_W), jnp.float32),
            pltpu.VMEM((2, TOK_PER_W), jnp.float32),
            pltpu.VMEM((2, TOK_PER_W), jnp.int32),
        ],
    )(_top2_sc_body)
    return f(gate_t)


def kernel(x, W):
    gate_t = _gate_tc(x, W)
    val_t, idx_t = _top2_sc(gate_t)
    return val_t.T, idx_t.T, gate_t.T


# speedup vs baseline: 1.0806x; 1.0806x over previous
import functools

import jax
import jax.numpy as jnp
from jax import lax
from jax.experimental import pallas as pl
from jax.experimental.pallas import tpu as pltpu
from jax.experimental.pallas import tpu_sc as plsc

N_TOK = 8192
HID = 2048
N_EXP = 16
BM = 1024

NW = 16
TOK_PER_W = N_TOK // NW


def _router_gate_body(xa_ref, xb_ref, w_ref, gate_ref):
    dn = (((1,), (1,)), ((), ()))
    logits = lax.dot_general(
        w_ref[:, : HID // 2], xa_ref[...],
        dimension_numbers=dn, preferred_element_type=jnp.float32,
    ) + lax.dot_general(
        w_ref[:, HID // 2 :], xb_ref[...],
        dimension_numbers=dn, preferred_element_type=jnp.float32,
    )
    m = jnp.max(logits, axis=0, keepdims=True)
    e = jnp.exp(logits - m)
    gate_ref[...] = e / jnp.sum(e, axis=0, keepdims=True)


def _gate_tc(x, w):
    return pl.pallas_call(
        _router_gate_body,
        grid=(N_TOK // BM,),
        in_specs=[
            pl.BlockSpec((BM, HID // 2), lambda i: (i, 0)),
            pl.BlockSpec((BM, HID // 2), lambda i: (i, 1)),
            pl.BlockSpec((N_EXP, HID), lambda i: (0, 0)),
        ],
        out_specs=pl.BlockSpec((N_EXP, BM), lambda i: (0, i)),
        out_shape=jax.ShapeDtypeStruct((N_EXP, N_TOK), jnp.float32),
    )(x, x, w)


def _top2_sc_body(gate_hbm, val_hbm, idx_hbm, gate_v, outv, outi):
    wid = lax.axis_index("s") * 1 + lax.axis_index("c")
    base = wid * TOK_PER_W

    pltpu.sync_copy(gate_hbm.at[:, pl.ds(base, TOK_PER_W)], gate_v)

    def grp_body(g, carry):
        sl = pl.ds(g * 16, 16)
        vals = [gate_v[e, sl] for e in range(N_EXP)]
        m1 = vals[0]
        i1 = jnp.zeros((16,), jnp.int32)
        for e in range(1, N_EXP):
            c = vals[e] > m1
            m1 = jnp.where(c, vals[e], m1)
            i1 = jnp.where(c, e, i1)
        m2 = jnp.full((16,), -1.0, jnp.float32)
        i2 = jnp.zeros((16,), jnp.int32)
        for e in range(N_EXP):
            ve = jnp.where(i1 == e, -1.0, vals[e])
            c = ve > m2
            m2 = jnp.where(c, ve, m2)
            i2 = jnp.where(c, e, i2)
        outv[0, sl] = m1
        outv[1, sl] = m2
        outi[0, sl] = i1
        outi[1, sl] = i2
        return carry

    lax.fori_loop(0, TOK_PER_W // 16, grp_body, 0, unroll=4)

    pltpu.sync_copy(outv, val_hbm.at[:, pl.ds(base, TOK_PER_W)])
    pltpu.sync_copy(outi, idx_hbm.at[:, pl.ds(base, TOK_PER_W)])


def _top2_sc(gate_t):
    mesh = plsc.VectorSubcoreMesh(core_axis_name="c", subcore_axis_name="s", num_cores=1)
    f = functools.partial(
        pl.kernel,
        out_type=[
            jax.ShapeDtypeStruct((2, N_TOK), jnp.float32),
            jax.ShapeDtypeStruct((2, N_TOK), jnp.int32),
        ],
        mesh=mesh,
        compiler_params=pltpu.CompilerParams(needs_layout_passes=False),
        scratch_types=[
            pltpu.VMEM((N_EXP, TOK_PER_W), jnp.float32),
            pltpu.VMEM((2, TOK_PER_W), jnp.float32),
            pltpu.VMEM((2, TOK_PER_W), jnp.int32),
        ],
    )(_top2_sc_body)
    return f(gate_t)


def kernel(x, W):
    gate_t = _gate_tc(x, W)
    val_t, idx_t = _top2_sc(gate_t)
    return val_t.T, idx_t.T, gate_t.T


# speedup vs baseline: 1.0890x; 1.0078x over previous
import functools

import jax
import jax.numpy as jnp
from jax import lax
from jax.experimental import pallas as pl
from jax.experimental.pallas import tpu as pltpu
from jax.experimental.pallas import tpu_sc as plsc

N_TOK = 8192
HID = 2048
N_EXP = 16
BM = 1024

NW = 16
TOK_PER_W = N_TOK // NW


def _router_gate_body(xa_ref, xb_ref, w_ref, gate_ref):
    dn = (((1,), (1,)), ((), ()))
    logits = lax.dot_general(
        w_ref[:, : HID // 2], xa_ref[...],
        dimension_numbers=dn, preferred_element_type=jnp.float32,
    ) + lax.dot_general(
        w_ref[:, HID // 2 :], xb_ref[...],
        dimension_numbers=dn, preferred_element_type=jnp.float32,
    )
    m = jnp.max(logits, axis=0, keepdims=True)
    e = jnp.exp(logits - m)
    gate_ref[...] = e / jnp.sum(e, axis=0, keepdims=True)


def _gate_tc(x, w):
    return pl.pallas_call(
        _router_gate_body,
        grid=(N_TOK // BM,),
        in_specs=[
            pl.BlockSpec((BM, HID // 2), lambda i: (i, 0)),
            pl.BlockSpec((BM, HID // 2), lambda i: (i, 1)),
            pl.BlockSpec((N_EXP, HID), lambda i: (0, 0)),
        ],
        out_specs=pl.BlockSpec((N_EXP, BM), lambda i: (0, i)),
        out_shape=jax.ShapeDtypeStruct((N_EXP, N_TOK), jnp.float32),
    )(x, x, w)


def _top2_sc_body(gate_hbm, val_hbm, idx_hbm, gate_v, outv, outi):
    wid = lax.axis_index("s") + lax.axis_index("c")
    base = wid * TOK_PER_W

    pltpu.sync_copy(gate_hbm.at[:, pl.ds(base, TOK_PER_W)], gate_v)

    def grp_body(g, carry):
        sl = pl.ds(g * 16, 16)
        vals = [gate_v[e, sl] for e in range(N_EXP)]
        m1 = vals[0]
        i1 = jnp.zeros((16,), jnp.int32)
        for e in range(1, N_EXP):
            c = vals[e] > m1
            m1 = jnp.where(c, vals[e], m1)
            i1 = jnp.where(c, e, i1)
        m2 = jnp.full((16,), -1.0, jnp.float32)
        i2 = jnp.zeros((16,), jnp.int32)
        for e in range(N_EXP):
            ve = jnp.where(i1 == e, -1.0, vals[e])
            c = ve > m2
            m2 = jnp.where(c, ve, m2)
            i2 = jnp.where(c, e, i2)
        outv[0, sl] = m1
        outv[1, sl] = m2
        outi[0, sl] = i1
        outi[1, sl] = i2
        return carry

    lax.fori_loop(0, TOK_PER_W // 16, grp_body, 0, unroll=2)

    pltpu.sync_copy(outv, val_hbm.at[:, pl.ds(base, TOK_PER_W)])
    pltpu.sync_copy(outi, idx_hbm.at[:, pl.ds(base, TOK_PER_W)])


def _top2_sc(gate_t):
    mesh = plsc.VectorSubcoreMesh(core_axis_name="c", subcore_axis_name="s", num_cores=1)
    f = functools.partial(
        pl.kernel,
        out_type=[
            jax.ShapeDtypeStruct((2, N_TOK), jnp.float32),
            jax.ShapeDtypeStruct((2, N_TOK), jnp.int32),
        ],
        mesh=mesh,
        compiler_params=pltpu.CompilerParams(needs_layout_passes=False),
        scratch_types=[
            pltpu.VMEM((N_EXP, TOK_PER_W), jnp.float32),
            pltpu.VMEM((2, TOK_PER_W), jnp.float32),
            pltpu.VMEM((2, TOK_PER_W), jnp.int32),
        ],
    )(_top2_sc_body)
    return f(gate_t)


def kernel(x, W):
    gate_t = _gate_tc(x, W)
    val_t, idx_t = _top2_sc(gate_t)
    return val_t.T, idx_t.T, gate_t.T
